# Initial kernel scaffold; baseline (speedup 1.0000x reference)
import jax
import jax.numpy as jnp
from jax.experimental import pallas as pl


def kernel(source, nnf):
    raise NotImplementedError("write your pallas kernel here")


# speedup vs baseline: 5.8762x; 5.8762x over previous
import functools

import jax
import jax.numpy as jnp
from jax import lax
from jax.experimental import pallas as pl
from jax.experimental.pallas import tpu as pltpu
from jax.experimental.pallas import tpu_sc as plsc

_NC, _NS, _L = 2, 16, 16
_NW = _NC * _NS
_K = 128


def _tec_body(H, W, HW, n_chunks,
              src_hbm, nnf0_hbm, nnf1_hbm, out_hbm,
              nnf0_v, nnf1_v,
              idx_a, idx_b, idx_c, idx_d,
              wa_v, wb_v, wc_v, wd_v,
              buf_a, buf_b, buf_c, buf_d, acc_v, sem):
  C = acc_v.shape[1]
  wid = lax.axis_index("s") * _NC + lax.axis_index("c")
  per_w = n_chunks * _K
  base0 = wid * per_w

  def chunk(ci, carry):
    base = base0 + ci * _K
    pltpu.sync_copy(nnf0_hbm.at[pl.ds(base, _K)], nnf0_v)
    pltpu.sync_copy(nnf1_hbm.at[pl.ds(base, _K)], nnf1_v)

    def grp(i, c2):
      sl = pl.ds(i * _L, _L)
      p = base + i * _L + lax.iota(jnp.int32, _L)
      wcoord = lax.rem(p, W)
      hcoord = lax.rem(lax.div(p, W), H)
      bcoord = lax.div(p, HW)
      wf = wcoord.astype(jnp.float32)
      hf = hcoord.astype(jnp.float32)
      g0 = jnp.clip((wf - (W // 2)) / W + nnf0_v[sl], -1.0, 1.0)
      g1 = jnp.clip((hf - (H // 2)) / H + nnf1_v[sl], -1.0, 1.0)
      x = (g0 + 1.0) * W / 2.0 - 0.5
      y = (g1 + 1.0) * H / 2.0 - 0.5
      tx = x.astype(jnp.int32)
      x0 = tx - jnp.where(x < tx.astype(jnp.float32), 1, 0)
      ty = y.astype(jnp.int32)
      y0 = ty - jnp.where(y < ty.astype(jnp.float32), 1, 0)
      x0f = x0.astype(jnp.float32)
      y0f = y0.astype(jnp.float32)
      wx0 = (x0f + 1.0) - x
      wx1 = x - x0f
      wy0 = (y0f + 1.0) - y
      wy1 = y - y0f
      vx0 = x0 >= 0
      vx1 = x0 <= (W - 2)
      vy0 = y0 >= 0
      vy1 = y0 <= (H - 2)
      zero = jnp.zeros_like(x)
      wa = jnp.where(vx0 & vy0, wx0 * wy0, zero)
      wb = jnp.where(vx0 & vy1, wx0 * wy1, zero)
      wc = jnp.where(vx1 & vy0, wx1 * wy0, zero)
      wd = jnp.where(vx1 & vy1, wx1 * wy1, zero)
      x0c = jnp.maximum(x0, 0)
      x1c = jnp.minimum(x0 + 1, W - 1)
      y0c = jnp.maximum(y0, 0)
      y1c = jnp.minimum(y0 + 1, H - 1)
      row_b = bcoord * HW
      r0 = row_b + y0c * W
      r1 = row_b + y1c * W
      idx_a[sl] = r0 + x0c
      idx_b[sl] = r1 + x0c
      idx_c[sl] = r0 + x1c
      idx_d[sl] = r1 + x1c
      wa_v[sl] = wa
      wb_v[sl] = wb
      wc_v[sl] = wc
      wd_v[sl] = wd
      return c2

    lax.fori_loop(0, _K // _L, grp, 0)

    da = pltpu.async_copy(src_hbm.at[idx_a], buf_a, sem)
    db = pltpu.async_copy(src_hbm.at[idx_b], buf_b, sem)
    dc = pltpu.async_copy(src_hbm.at[idx_c], buf_c, sem)
    dd = pltpu.async_copy(src_hbm.at[idx_d], buf_d, sem)
    da.wait()
    db.wait()
    dc.wait()
    dd.wait()

    def blend(i, c2):
      sl = pl.ds(i * _L, _L)
      wa16 = wa_v[sl]
      wb16 = wb_v[sl]
      wc16 = wc_v[sl]
      wd16 = wd_v[sl]
      for j in range(_L):
        k = i * _L + j
        wa = wa16[j]
        wb = wb16[j]
        wc = wc16[j]
        wd = wd16[j]
        for g in range(C // _L):
          s2 = pl.ds(g * _L, _L)
          acc_v[k, s2] = (buf_a[k, s2] * wa + buf_b[k, s2] * wb
                          + buf_c[k, s2] * wc + buf_d[k, s2] * wd)
      return c2

    lax.fori_loop(0, _K // _L, blend, 0)
    pltpu.sync_copy(acc_v, out_hbm.at[pl.ds(base, _K)])
    return carry

  lax.fori_loop(0, n_chunks, chunk, 0)


def kernel(source, nnf):
  B, C, H, W = source.shape
  HW = H * W
  N = B * HW
  n_chunks = N // (_NW * _K)
  src_rows = source.transpose(0, 2, 3, 1).reshape(N, C)
  nnf0 = nnf[:, 0].reshape(N)
  nnf1 = nnf[:, 1].reshape(N)

  mesh = plsc.VectorSubcoreMesh(core_axis_name="c", subcore_axis_name="s",
                                num_cores=_NC, num_subcores=_NS)
  body = functools.partial(_tec_body, H, W, HW, n_chunks)
  out_rows = pl.kernel(
      body,
      out_type=jax.ShapeDtypeStruct((N, C), jnp.float32),
      mesh=mesh,
      compiler_params=pltpu.CompilerParams(use_tc_tiling_on_sc=False),
      scratch_types=[
          pltpu.VMEM((_K,), jnp.float32),
          pltpu.VMEM((_K,), jnp.float32),
          pltpu.VMEM((_K,), jnp.int32),
          pltpu.VMEM((_K,), jnp.int32),
          pltpu.VMEM((_K,), jnp.int32),
          pltpu.VMEM((_K,), jnp.int32),
          pltpu.VMEM((_K,), jnp.float32),
          pltpu.VMEM((_K,), jnp.float32),
          pltpu.VMEM((_K,), jnp.float32),
          pltpu.VMEM((_K,), jnp.float32),
          pltpu.VMEM((_K, C), jnp.float32),
          pltpu.VMEM((_K, C), jnp.float32),
          pltpu.VMEM((_K, C), jnp.float32),
          pltpu.VMEM((_K, C), jnp.float32),
          pltpu.VMEM((_K, C), jnp.float32),
          pltpu.SemaphoreType.DMA,
      ],
  )(src_rows, nnf0, nnf1)
  return out_rows.reshape(B, H, W, C).transpose(0, 3, 1, 2)


# speedup vs baseline: 7.1204x; 1.2117x over previous
import functools

import jax
import jax.numpy as jnp
from jax import lax
from jax.experimental import pallas as pl
from jax.experimental.pallas import tpu as pltpu
from jax.experimental.pallas import tpu_sc as plsc

_NC, _NS, _L = 2, 16, 16
_NW = _NC * _NS
_K = 128


def _tec_body(H, W, HW, n_chunks,
              src_hbm, nnf0_hbm, nnf1_hbm, out_hbm,
              nnf0_v, nnf1_v, idx_v, w_v, bufs, acc_v,
              sem_n, sem_g, sem_o):
  C = acc_v.shape[1]
  wid = lax.axis_index("s") * _NC + lax.axis_index("c")
  per_w = n_chunks * _K
  base0 = wid * per_w

  def nnf_load(ci, s):
    base = base0 + ci * _K
    pltpu.async_copy(nnf0_hbm.at[pl.ds(base, _K)], nnf0_v[s], sem_n[s])
    pltpu.async_copy(nnf1_hbm.at[pl.ds(base, _K)], nnf1_v[s], sem_n[s])

  def nnf_wait(s):
    pltpu.make_async_copy(nnf0_hbm.at[pl.ds(0, _K)], nnf0_v[s], sem_n[s]).wait()
    pltpu.make_async_copy(nnf1_hbm.at[pl.ds(0, _K)], nnf1_v[s], sem_n[s]).wait()

  def prep(ci, s):
    base = base0 + ci * _K

    def grp(i, c2):
      sl = pl.ds(i * _L, _L)
      p = base + i * _L + lax.iota(jnp.int32, _L)
      wcoord = lax.rem(p, W)
      hcoord = lax.rem(lax.div(p, W), H)
      bcoord = lax.div(p, HW)
      wf = wcoord.astype(jnp.float32)
      hf = hcoord.astype(jnp.float32)
      g0 = jnp.clip((wf - (W // 2)) / W + nnf0_v[s][sl], -1.0, 1.0)
      g1 = jnp.clip((hf - (H // 2)) / H + nnf1_v[s][sl], -1.0, 1.0)
      x = (g0 + 1.0) * W / 2.0 - 0.5
      y = (g1 + 1.0) * H / 2.0 - 0.5
      tx = x.astype(jnp.int32)
      x0 = tx - jnp.where(x < tx.astype(jnp.float32), 1, 0)
      ty = y.astype(jnp.int32)
      y0 = ty - jnp.where(y < ty.astype(jnp.float32), 1, 0)
      x0f = x0.astype(jnp.float32)
      y0f = y0.astype(jnp.float32)
      wx0 = (x0f + 1.0) - x
      wx1 = x - x0f
      wy0 = (y0f + 1.0) - y
      wy1 = y - y0f
      vx0 = x0 >= 0
      vx1 = x0 <= (W - 2)
      vy0 = y0 >= 0
      vy1 = y0 <= (H - 2)
      zero = jnp.zeros_like(x)
      wa = jnp.where(vx0 & vy0, wx0 * wy0, zero)
      wb = jnp.where(vx0 & vy1, wx0 * wy1, zero)
      wc = jnp.where(vx1 & vy0, wx1 * wy0, zero)
      wd = jnp.where(vx1 & vy1, wx1 * wy1, zero)
      x0c = jnp.maximum(x0, 0)
      x1c = jnp.minimum(x0 + 1, W - 1)
      y0c = jnp.maximum(y0, 0)
      y1c = jnp.minimum(y0 + 1, H - 1)
      row_b = bcoord * HW
      r0 = row_b + y0c * W
      r1 = row_b + y1c * W
      idx_v[s][0][sl] = r0 + x0c
      idx_v[s][1][sl] = r1 + x0c
      idx_v[s][2][sl] = r0 + x1c
      idx_v[s][3][sl] = r1 + x1c
      w_v[s][0][sl] = wa
      w_v[s][1][sl] = wb
      w_v[s][2][sl] = wc
      w_v[s][3][sl] = wd
      return c2

    lax.fori_loop(0, _K // _L, grp, 0)
    for q in range(4):
      pltpu.async_copy(src_hbm.at[idx_v[s][q]], bufs[s][q], sem_g[s])

  def gather_wait(s):
    for q in range(4):
      pltpu.make_async_copy(src_hbm.at[idx_v[s][q]], bufs[s][q],
                            sem_g[s]).wait()

  def out_wait():
    pltpu.make_async_copy(acc_v, out_hbm.at[pl.ds(0, _K)], sem_o).wait()

  def emit(ci, s):
    base = base0 + ci * _K

    def blend(i, c2):
      sl = pl.ds(i * _L, _L)
      w16 = [w_v[s][q][sl] for q in range(4)]
      for j in range(_L):
        k = i * _L + j
        wa, wb, wc, wd = (w16[q][j] for q in range(4))
        for g in range(C // _L):
          s2 = pl.ds(g * _L, _L)
          acc_v[k, s2] = (bufs[s][0][k, s2] * wa + bufs[s][1][k, s2] * wb
                          + bufs[s][2][k, s2] * wc + bufs[s][3][k, s2] * wd)
      return c2

    lax.fori_loop(0, _K // _L, blend, 0)
    pltpu.async_copy(acc_v, out_hbm.at[pl.ds(base, _K)], sem_o)

  nnf_load(0, 0)
  nnf_wait(0)
  prep(0, 0)
  nnf_load(1, 1)

  def pair(t, carry):
    for par in range(2):
      i = t + par
      s = par

      @pl.when(i + 2 < n_chunks)
      def _():
        nnf_load(i + 2, s)

      @pl.when(i + 1 < n_chunks)
      def _():
        nnf_wait(1 - s)
        prep(i + 1, 1 - s)

      gather_wait(s)

      @pl.when(i >= 1)
      def _():
        out_wait()

      emit(i, s)
    return carry

  lax.fori_loop(0, n_chunks // 2, lambda t2, c: pair(t2 * 2, c), 0)
  out_wait()


def kernel(source, nnf):
  B, C, H, W = source.shape
  HW = H * W
  N = B * HW
  n_chunks = N // (_NW * _K)
  src_rows = source.transpose(0, 2, 3, 1).reshape(N, C)
  nnf0 = nnf[:, 0].reshape(N)
  nnf1 = nnf[:, 1].reshape(N)

  mesh = plsc.VectorSubcoreMesh(core_axis_name="c", subcore_axis_name="s",
                                num_cores=_NC, num_subcores=_NS)
  body = functools.partial(_tec_body, H, W, HW, n_chunks)
  out_rows = pl.kernel(
      body,
      out_type=jax.ShapeDtypeStruct((N, C), jnp.float32),
      mesh=mesh,
      compiler_params=pltpu.CompilerParams(use_tc_tiling_on_sc=False),
      scratch_types=[
          [pltpu.VMEM((_K,), jnp.float32) for _ in range(2)],
          [pltpu.VMEM((_K,), jnp.float32) for _ in range(2)],
          [[pltpu.VMEM((_K,), jnp.int32) for _ in range(4)]
           for _ in range(2)],
          [[pltpu.VMEM((_K,), jnp.float32) for _ in range(4)]
           for _ in range(2)],
          [[pltpu.VMEM((_K, C), jnp.float32) for _ in range(4)]
           for _ in range(2)],
          pltpu.VMEM((_K, C), jnp.float32),
          [pltpu.SemaphoreType.DMA for _ in range(2)],
          [pltpu.SemaphoreType.DMA for _ in range(2)],
          pltpu.SemaphoreType.DMA,
      ],
  )(src_rows, nnf0, nnf1)
  return out_rows.reshape(B, H, W, C).transpose(0, 3, 1, 2)
